# Initial kernel scaffold; baseline (speedup 1.0000x reference)
import jax
import jax.numpy as jnp
from jax.experimental import pallas as pl


def kernel(node, adj, weight, mask, W_embed, W_self1, W_msg1, W_self2, W_msg2, centroids, W_out, b_out):
    raise NotImplementedError("write your pallas kernel here")


# speedup vs baseline: 78.9781x; 78.9781x over previous
import jax
import jax.numpy as jnp
from jax import lax
from jax.experimental import pallas as pl
from jax.experimental.pallas import tpu as pltpu

_B, _N, _K, _F, _D, _C, _NC = 32, 512, 32, 256, 256, 100, 10
_CP = 128


def _tc_body(mask_ref, node_ref, adj_ref, w_ref, we_ref, ws1_ref, wm1_ref,
             ws2_ref, wm2_ref, cent_ref, wout_ref, bout_ref, out_ref):
    b = pl.program_id(0)
    mi = mask_ref[b, 0]
    m = mi.astype(jnp.float32)

    node = node_ref[0]
    adj = adj_ref[0]
    w = w_ref[0]

    iota_n = lax.broadcasted_iota(jnp.int32, (_N, _N), 1)
    acc = jnp.zeros((_N, _N), jnp.float32)
    for k in range(_K):
        ak = adj[:, k][:, None]
        wk = w[:, k][:, None]
        acc = acc + jnp.where(ak == iota_n, wk, 0.0)

    iota_col = lax.broadcasted_iota(jnp.int32, (_N, 1), 0)
    nm = (iota_col < mi).astype(jnp.float32)

    h = jnp.dot(node, we_ref[...], preferred_element_type=jnp.float32) * nm
    for ws, wm in ((ws1_ref, wm1_ref), (ws2_ref, wm2_ref)):
        agg = jnp.dot(acc, h, preferred_element_type=jnp.float32)
        h = jnp.maximum(
            jnp.dot(h, ws[...], preferred_element_type=jnp.float32)
            + jnp.dot(agg, wm[...], preferred_element_type=jnp.float32),
            0.0) * nm

    cent = cent_ref[...]
    x2 = jnp.sum(h * h, axis=1, keepdims=True)
    c2 = jnp.sum(cent * cent, axis=0, keepdims=True)
    d2 = x2 + c2 - 2.0 * jnp.dot(h, cent, preferred_element_type=jnp.float32)
    dist = jnp.sqrt(jnp.maximum(d2, 1e-6))
    gr = jnp.sum(dist * nm, axis=0, keepdims=True) / m
    out_ref[0] = jnp.dot(gr, wout_ref[...],
                         preferred_element_type=jnp.float32) + bout_ref[...]


def kernel(node, adj, weight, mask, W_embed, W_self1, W_msg1, W_self2,
           W_msg2, centroids, W_out, b_out):
    adj32 = adj.astype(jnp.int32)
    mask32 = mask.astype(jnp.int32)
    weT = W_embed.T
    ws1T, wm1T = W_self1.T, W_msg1.T
    ws2T, wm2T = W_self2.T, W_msg2.T
    centP = jnp.zeros((_D, _CP), jnp.float32).at[:, :_C].set(centroids.T)
    woutP = jnp.zeros((_CP, _CP), jnp.float32).at[:_C, :_NC].set(W_out.T)
    boutP = jnp.zeros((1, _CP), jnp.float32).at[0, :_NC].set(b_out)

    rep = lambda b: (0, 0)
    out = pl.pallas_call(
        _tc_body,
        grid=(_B,),
        in_specs=[
            pl.BlockSpec(memory_space=pltpu.SMEM),
            pl.BlockSpec((1, _N, _F), lambda b: (b, 0, 0)),
            pl.BlockSpec((1, _N, _K), lambda b: (b, 0, 0)),
            pl.BlockSpec((1, _N, _K), lambda b: (b, 0, 0)),
            pl.BlockSpec((_F, _D), rep),
            pl.BlockSpec((_D, _D), rep),
            pl.BlockSpec((_D, _D), rep),
            pl.BlockSpec((_D, _D), rep),
            pl.BlockSpec((_D, _D), rep),
            pl.BlockSpec((_D, _CP), rep),
            pl.BlockSpec((_CP, _CP), rep),
            pl.BlockSpec((1, _CP), rep),
        ],
        out_specs=pl.BlockSpec((1, 1, _CP), lambda b: (b, 0, 0)),
        out_shape=jax.ShapeDtypeStruct((_B, 1, _CP), jnp.float32),
    )(mask32, node, adj32, weight, weT, ws1T, wm1T, ws2T, wm2T,
      centP, woutP, boutP)
    return out[:, 0, :_NC]


# speedup vs baseline: 108.3415x; 1.3718x over previous
import functools

import jax
import jax.numpy as jnp
from jax import lax
from jax.experimental import pallas as pl
from jax.experimental.pallas import tpu as pltpu
from jax.experimental.pallas import tpu_sc as plsc

_B, _N, _K, _F, _D, _C, _NC = 32, 512, 32, 256, 256, 100, 10
_CP = 128
_RC = 128
_NCORES, _NSUB, _L = 2, 16, 16


def _sc_build_body(adj_hbm, w_hbm, a_hbm, adj_v, w_v, chunk):
    b = lax.axis_index("s") * _NCORES + lax.axis_index("c")
    pltpu.sync_copy(adj_hbm.at[b], adj_v)
    pltpu.sync_copy(w_hbm.at[b], w_v)
    zero16 = jnp.zeros((_L,), jnp.float32)
    iota16 = lax.iota(jnp.int32, _L)
    for c in range(_N // _RC):
        def zrow(i, carry):
            chunk[pl.ds(i * _L, _L)] = zero16
            return carry
        lax.fori_loop(0, _RC * _N // _L, zrow, 0)

        def srow(rg, carry):
            row0 = rg * _L
            base = (iota16 + row0) * _N
            for k in range(_K):
                cols = adj_v[k, pl.ds(c * _RC + row0, _L)]
                vals = w_v[k, pl.ds(c * _RC + row0, _L)]
                plsc.addupdate_scatter(chunk, [base + cols], vals)
            return carry
        lax.fori_loop(0, _RC // _L, srow, 0)
        pltpu.sync_copy(chunk, a_hbm.at[b, pl.ds(c * _RC * _N, _RC * _N)])


_sc_build = functools.partial(
    pl.kernel,
    out_type=jax.ShapeDtypeStruct((_B, _N * _N), jnp.float32),
    mesh=plsc.VectorSubcoreMesh(core_axis_name="c", subcore_axis_name="s"),
    compiler_params=pltpu.CompilerParams(needs_layout_passes=False),
    scratch_types=[
        pltpu.VMEM((_K, _N), jnp.int32),
        pltpu.VMEM((_K, _N), jnp.float32),
        pltpu.VMEM((_RC * _N,), jnp.float32),
    ],
)(_sc_build_body)


def _tc_body(mask_ref, node_ref, a_ref, we_ref, ws1_ref, wm1_ref,
             ws2_ref, wm2_ref, cent_ref, wout_ref, bout_ref, out_ref):
    b = pl.program_id(0)
    mi = mask_ref[b, 0]
    m = mi.astype(jnp.float32)

    node = node_ref[0]
    acc = a_ref[0]

    iota_col = lax.broadcasted_iota(jnp.int32, (_N, 1), 0)
    nm = (iota_col < mi).astype(jnp.float32)

    h = jnp.dot(node, we_ref[...], preferred_element_type=jnp.float32) * nm
    for ws, wm in ((ws1_ref, wm1_ref), (ws2_ref, wm2_ref)):
        agg = jnp.dot(acc, h, preferred_element_type=jnp.float32)
        h = jnp.maximum(
            jnp.dot(h, ws[...], preferred_element_type=jnp.float32)
            + jnp.dot(agg, wm[...], preferred_element_type=jnp.float32),
            0.0) * nm

    cent = cent_ref[...]
    x2 = jnp.sum(h * h, axis=1, keepdims=True)
    c2 = jnp.sum(cent * cent, axis=0, keepdims=True)
    d2 = x2 + c2 - 2.0 * jnp.dot(h, cent, preferred_element_type=jnp.float32)
    dist = jnp.sqrt(jnp.maximum(d2, 1e-6))
    gr = jnp.sum(dist * nm, axis=0, keepdims=True) / m
    out_ref[0] = jnp.dot(gr, wout_ref[...],
                         preferred_element_type=jnp.float32) + bout_ref[...]


def kernel(node, adj, weight, mask, W_embed, W_self1, W_msg1, W_self2,
           W_msg2, centroids, W_out, b_out):
    adjT = jnp.swapaxes(adj.astype(jnp.int32), 1, 2)
    wT = jnp.swapaxes(weight, 1, 2)
    mask32 = mask.astype(jnp.int32)
    weT = W_embed.T
    ws1T, wm1T = W_self1.T, W_msg1.T
    ws2T, wm2T = W_self2.T, W_msg2.T
    centP = jnp.zeros((_D, _CP), jnp.float32).at[:, :_C].set(centroids.T)
    woutP = jnp.zeros((_CP, _CP), jnp.float32).at[:_C, :_NC].set(W_out.T)
    boutP = jnp.zeros((1, _CP), jnp.float32).at[0, :_NC].set(b_out)

    a_dense = _sc_build(adjT, wT).reshape(_B, _N, _N)

    rep = lambda b: (0, 0)
    out = pl.pallas_call(
        _tc_body,
        grid=(_B,),
        in_specs=[
            pl.BlockSpec(memory_space=pltpu.SMEM),
            pl.BlockSpec((1, _N, _F), lambda b: (b, 0, 0)),
            pl.BlockSpec((1, _N, _N), lambda b: (b, 0, 0)),
            pl.BlockSpec((_F, _D), rep),
            pl.BlockSpec((_D, _D), rep),
            pl.BlockSpec((_D, _D), rep),
            pl.BlockSpec((_D, _D), rep),
            pl.BlockSpec((_D, _D), rep),
            pl.BlockSpec((_D, _CP), rep),
            pl.BlockSpec((_CP, _CP), rep),
            pl.BlockSpec((1, _CP), rep),
        ],
        out_specs=pl.BlockSpec((1, 1, _CP), lambda b: (b, 0, 0)),
        out_shape=jax.ShapeDtypeStruct((_B, 1, _CP), jnp.float32),
    )(mask32, node, a_dense, weT, ws1T, wm1T, ws2T, wm2T,
      centP, woutP, boutP)
    return out[:, 0, :_NC]


# speedup vs baseline: 122.6940x; 1.1325x over previous
import functools

import jax
import jax.numpy as jnp
from jax import lax
from jax.experimental import pallas as pl
from jax.experimental.pallas import tpu as pltpu
from jax.experimental.pallas import tpu_sc as plsc

_B, _N, _K, _F, _D, _C, _NC = 32, 512, 32, 256, 256, 100, 10
_CP = 128
_RC = 64
_NCORES, _NSUB, _L = 2, 16, 16


def _sc_build_body(adj_hbm, w_hbm, a_hbm, adj_v, w_v, chunk0, chunk1,
                   sem0, sem1):
    b = lax.axis_index("s") * _NCORES + lax.axis_index("c")
    pltpu.sync_copy(adj_hbm.at[b], adj_v)
    pltpu.sync_copy(w_hbm.at[b], w_v)
    zero16 = jnp.zeros((_L,), jnp.float32)
    iota16 = lax.iota(jnp.int32, _L)
    iotaK = iota16 * _K
    chunks = (chunk0, chunk1)
    sems = (sem0, sem1)
    copies = []
    for c in range(_N // _RC):
        chunk = chunks[c % 2]
        if c >= 2:
            copies[c - 2].wait()

        def zrow(i, carry):
            for j in range(_N // _L):
                chunk[pl.ds(i * _N + j * _L, _L)] = zero16
            return carry
        lax.fori_loop(0, _RC, zrow, 0)

        def srow(rg, carry):
            row0 = c * _RC + rg * _L
            gbase = iotaK + row0 * _K
            sbase = (iota16 + rg * _L) * _N
            for k in range(_K):
                cols = plsc.load_gather(adj_v, [gbase + k])
                vals = plsc.load_gather(w_v, [gbase + k])
                plsc.addupdate_scatter(chunk, [sbase + cols], vals)
            return carry
        lax.fori_loop(0, _RC // _L, srow, 0)

        cp = pltpu.make_async_copy(
            chunk, a_hbm.at[b, pl.ds(c * _RC * _N, _RC * _N)], sems[c % 2])
        cp.start()
        copies.append(cp)
    copies[-2].wait()
    copies[-1].wait()


_sc_build = functools.partial(
    pl.kernel,
    out_type=jax.ShapeDtypeStruct((_B, _N * _N), jnp.float32),
    mesh=plsc.VectorSubcoreMesh(core_axis_name="c", subcore_axis_name="s"),
    compiler_params=pltpu.CompilerParams(needs_layout_passes=False),
    scratch_types=[
        pltpu.VMEM((_N * _K,), jnp.int32),
        pltpu.VMEM((_N * _K,), jnp.float32),
        pltpu.VMEM((_RC * _N,), jnp.float32),
        pltpu.VMEM((_RC * _N,), jnp.float32),
        pltpu.SemaphoreType.DMA,
        pltpu.SemaphoreType.DMA,
    ],
)(_sc_build_body)


def _tc_body(mask_ref, node_ref, a_ref, we_ref, ws1_ref, wm1_ref,
             ws2_ref, wm2_ref, cent_ref, wout_ref, bout_ref, out_ref):
    b = pl.program_id(0)
    mi = mask_ref[b, 0]
    m = mi.astype(jnp.float32)

    node = node_ref[0]
    acc = a_ref[0]

    iota_col = lax.broadcasted_iota(jnp.int32, (_N, 1), 0)
    nm = (iota_col < mi).astype(jnp.float32)

    h = jnp.dot(node, we_ref[...], preferred_element_type=jnp.float32) * nm
    for ws, wm in ((ws1_ref, wm1_ref), (ws2_ref, wm2_ref)):
        agg = jnp.dot(acc, h, preferred_element_type=jnp.float32)
        h = jnp.maximum(
            jnp.dot(h, ws[...], preferred_element_type=jnp.float32)
            + jnp.dot(agg, wm[...], preferred_element_type=jnp.float32),
            0.0) * nm

    cent = cent_ref[...]
    x2 = jnp.sum(h * h, axis=1, keepdims=True)
    c2 = jnp.sum(cent * cent, axis=0, keepdims=True)
    d2 = x2 + c2 - 2.0 * jnp.dot(h, cent, preferred_element_type=jnp.float32)
    dist = jnp.sqrt(jnp.maximum(d2, 1e-6))
    gr = jnp.sum(dist * nm, axis=0, keepdims=True) / m
    out_ref[0] = jnp.dot(gr, wout_ref[...],
                         preferred_element_type=jnp.float32) + bout_ref[...]


def kernel(node, adj, weight, mask, W_embed, W_self1, W_msg1, W_self2,
           W_msg2, centroids, W_out, b_out):
    adj_flat = adj.astype(jnp.int32).reshape(_B, _N * _K)
    w_flat = weight.reshape(_B, _N * _K)
    mask32 = mask.astype(jnp.int32)
    weT = W_embed.T
    ws1T, wm1T = W_self1.T, W_msg1.T
    ws2T, wm2T = W_self2.T, W_msg2.T
    centP = jnp.zeros((_D, _CP), jnp.float32).at[:, :_C].set(centroids.T)
    woutP = jnp.zeros((_CP, _CP), jnp.float32).at[:_C, :_NC].set(W_out.T)
    boutP = jnp.zeros((1, _CP), jnp.float32).at[0, :_NC].set(b_out)

    a_dense = _sc_build(adj_flat, w_flat).reshape(_B, _N, _N)

    rep = lambda b: (0, 0)
    out = pl.pallas_call(
        _tc_body,
        grid=(_B,),
        in_specs=[
            pl.BlockSpec(memory_space=pltpu.SMEM),
            pl.BlockSpec((1, _N, _F), lambda b: (b, 0, 0)),
            pl.BlockSpec((1, _N, _N), lambda b: (b, 0, 0)),
            pl.BlockSpec((_F, _D), rep),
            pl.BlockSpec((_D, _D), rep),
            pl.BlockSpec((_D, _D), rep),
            pl.BlockSpec((_D, _D), rep),
            pl.BlockSpec((_D, _D), rep),
            pl.BlockSpec((_D, _CP), rep),
            pl.BlockSpec((_CP, _CP), rep),
            pl.BlockSpec((1, _CP), rep),
        ],
        out_specs=pl.BlockSpec((1, 1, _CP), lambda b: (b, 0, 0)),
        out_shape=jax.ShapeDtypeStruct((_B, 1, _CP), jnp.float32),
    )(mask32, node, a_dense, weT, ws1T, wm1T, ws2T, wm2T,
      centP, woutP, boutP)
    return out[:, 0, :_NC]


# speedup vs baseline: 140.4347x; 1.1446x over previous
import functools

import jax
import jax.numpy as jnp
from jax import lax
from jax.experimental import pallas as pl
from jax.experimental.pallas import tpu as pltpu
from jax.experimental.pallas import tpu_sc as plsc

_B, _N, _K, _F, _D, _C, _NC = 32, 512, 32, 256, 256, 100, 10
_CP = 128
_RC = 64
_NCORES, _NSUB, _L = 2, 16, 16


def _sc_build_body(adj_hbm, w_hbm, a_hbm, adj_v, w_v, chunk0, chunk1,
                   sem0, sem1):
    b = lax.axis_index("s") * _NCORES + lax.axis_index("c")
    pltpu.sync_copy(adj_hbm.at[b], adj_v)
    pltpu.sync_copy(w_hbm.at[b], w_v)
    zero16 = jnp.zeros((_L,), jnp.float32)
    iota16 = lax.iota(jnp.int32, _L)
    iotaK = iota16 * _K
    chunks = (chunk0, chunk1)
    sems = (sem0, sem1)
    copies = []
    for c in range(_N // _RC):
        chunk = chunks[c % 2]
        if c >= 2:
            copies[c - 2].wait()

        def zrow(i, carry):
            for j in range(_N // _L):
                chunk[i, pl.ds(j * _L, _L)] = zero16
            return carry
        lax.fori_loop(0, _RC, zrow, 0)

        def srow(rg, carry):
            row0 = c * _RC + rg * _L
            gbase = iotaK + row0 * _K
            rows = iota16 + rg * _L
            for k in range(_K):
                cols = plsc.load_gather(adj_v, [gbase + k])
                vals = plsc.load_gather(w_v, [gbase + k])
                plsc.addupdate_scatter(chunk, [rows, cols], vals)
            return carry
        lax.fori_loop(0, _RC // _L, srow, 0)

        cp = pltpu.make_async_copy(
            chunk, a_hbm.at[b, pl.ds(c * _RC, _RC)], sems[c % 2])
        cp.start()
        copies.append(cp)
    copies[-2].wait()
    copies[-1].wait()


_sc_build = functools.partial(
    pl.kernel,
    out_type=jax.ShapeDtypeStruct((_B, _N, _N), jnp.float32),
    mesh=plsc.VectorSubcoreMesh(core_axis_name="c", subcore_axis_name="s"),
    compiler_params=pltpu.CompilerParams(needs_layout_passes=False),
    scratch_types=[
        pltpu.VMEM((_N * _K,), jnp.int32),
        pltpu.VMEM((_N * _K,), jnp.float32),
        pltpu.VMEM((_RC, _N), jnp.float32),
        pltpu.VMEM((_RC, _N), jnp.float32),
        pltpu.SemaphoreType.DMA,
        pltpu.SemaphoreType.DMA,
    ],
)(_sc_build_body)


def _tc_body(mask_ref, node_ref, a_ref, we_ref, ws1_ref, wm1_ref,
             ws2_ref, wm2_ref, cent_ref, wout_ref, bout_ref, out_ref):
    b = pl.program_id(0)
    mi = mask_ref[b, 0]
    m = mi.astype(jnp.float32)

    node = node_ref[0]
    acc = a_ref[0]

    iota_col = lax.broadcasted_iota(jnp.int32, (_N, 1), 0)
    nm = (iota_col < mi).astype(jnp.float32)

    h = jnp.dot(node, we_ref[...], preferred_element_type=jnp.float32) * nm
    for ws, wm in ((ws1_ref, wm1_ref), (ws2_ref, wm2_ref)):
        agg = jnp.dot(acc, h, preferred_element_type=jnp.float32)
        h = jnp.maximum(
            jnp.dot(h, ws[...], preferred_element_type=jnp.float32)
            + jnp.dot(agg, wm[...], preferred_element_type=jnp.float32),
            0.0) * nm

    cent = cent_ref[...]
    x2 = jnp.sum(h * h, axis=1, keepdims=True)
    c2 = jnp.sum(cent * cent, axis=0, keepdims=True)
    d2 = x2 + c2 - 2.0 * jnp.dot(h, cent, preferred_element_type=jnp.float32)
    dist = jnp.sqrt(jnp.maximum(d2, 1e-6))
    gr = jnp.sum(dist * nm, axis=0, keepdims=True) / m
    out_ref[0] = jnp.dot(gr, wout_ref[...],
                         preferred_element_type=jnp.float32) + bout_ref[...]


def kernel(node, adj, weight, mask, W_embed, W_self1, W_msg1, W_self2,
           W_msg2, centroids, W_out, b_out):
    adj_flat = adj.astype(jnp.int32).reshape(_B, _N * _K)
    w_flat = weight.reshape(_B, _N * _K)
    mask32 = mask.astype(jnp.int32)
    weT = W_embed.T
    ws1T, wm1T = W_self1.T, W_msg1.T
    ws2T, wm2T = W_self2.T, W_msg2.T
    centP = jnp.zeros((_D, _CP), jnp.float32).at[:, :_C].set(centroids.T)
    woutP = jnp.zeros((_CP, _CP), jnp.float32).at[:_C, :_NC].set(W_out.T)
    boutP = jnp.zeros((1, _CP), jnp.float32).at[0, :_NC].set(b_out)

    a_dense = _sc_build(adj_flat, w_flat)

    rep = lambda b: (0, 0)
    out = pl.pallas_call(
        _tc_body,
        grid=(_B,),
        in_specs=[
            pl.BlockSpec(memory_space=pltpu.SMEM),
            pl.BlockSpec((1, _N, _F), lambda b: (b, 0, 0)),
            pl.BlockSpec((1, _N, _N), lambda b: (b, 0, 0)),
            pl.BlockSpec((_F, _D), rep),
            pl.BlockSpec((_D, _D), rep),
            pl.BlockSpec((_D, _D), rep),
            pl.BlockSpec((_D, _D), rep),
            pl.BlockSpec((_D, _D), rep),
            pl.BlockSpec((_D, _CP), rep),
            pl.BlockSpec((_CP, _CP), rep),
            pl.BlockSpec((1, _CP), rep),
        ],
        out_specs=pl.BlockSpec((1, 1, _CP), lambda b: (b, 0, 0)),
        out_shape=jax.ShapeDtypeStruct((_B, 1, _CP), jnp.float32),
    )(mask32, node, a_dense, weT, ws1T, wm1T, ws2T, wm2T,
      centP, woutP, boutP)
    return out[:, 0, :_NC]


# speedup vs baseline: 164.7102x; 1.1729x over previous
import functools

import jax
import jax.numpy as jnp
from jax import lax
from jax.experimental import pallas as pl
from jax.experimental.pallas import tpu as pltpu
from jax.experimental.pallas import tpu_sc as plsc

_B, _N, _K, _F, _D, _C, _NC = 32, 512, 32, 256, 256, 100, 10
_CP = 128
_G = 4
_RC = 64
_NCORES, _NSUB, _L = 2, 16, 16


def _sc_build_body(adj_hbm, w_hbm, a_hbm, adj_v, w_v, chunk0, chunk1,
                   sem0, sem1):
    b = lax.axis_index("s") * _NCORES + lax.axis_index("c")
    pltpu.sync_copy(adj_hbm.at[b], adj_v)
    pltpu.sync_copy(w_hbm.at[b], w_v)
    zero16 = jnp.zeros((_L,), jnp.float32)
    iota16 = lax.iota(jnp.int32, _L)
    iotaK = iota16 * _K
    chunks = (chunk0, chunk1)
    sems = (sem0, sem1)
    copies = []
    for c in range(_N // _RC):
        chunk = chunks[c % 2]
        if c >= 2:
            copies[c - 2].wait()

        def zrow(i, carry):
            for j in range(_N // _L):
                chunk[i, pl.ds(j * _L, _L)] = zero16
            return carry
        lax.fori_loop(0, _RC, zrow, 0)

        def srow(rg, carry):
            row0 = c * _RC + rg * _L
            gbase = iotaK + row0 * _K
            rows = iota16 + rg * _L
            for k in range(_K):
                cols = plsc.load_gather(adj_v, [gbase + k])
                vals = plsc.load_gather(w_v, [gbase + k])
                plsc.addupdate_scatter(chunk, [rows, cols], vals)
            return carry
        lax.fori_loop(0, _RC // _L, srow, 0)

        cp = pltpu.make_async_copy(
            chunk, a_hbm.at[b, pl.ds(c * _RC, _RC)], sems[c % 2])
        cp.start()
        copies.append(cp)
    copies[-2].wait()
    copies[-1].wait()


def _sc_build(adj_flat, w_flat):
    built = functools.partial(
        pl.kernel,
        out_type=jax.ShapeDtypeStruct((_B, _N, _N), jnp.float32),
        mesh=plsc.VectorSubcoreMesh(core_axis_name="c", subcore_axis_name="s",
                                    num_cores=_NCORES, num_subcores=_NSUB),
        compiler_params=pltpu.CompilerParams(needs_layout_passes=False),
        scratch_types=[
            pltpu.VMEM((_N * _K,), jnp.int32),
            pltpu.VMEM((_N * _K,), jnp.float32),
            pltpu.VMEM((_RC, _N), jnp.float32),
            pltpu.VMEM((_RC, _N), jnp.float32),
            pltpu.SemaphoreType.DMA,
            pltpu.SemaphoreType.DMA,
        ],
    )(_sc_build_body)
    return built(adj_flat, w_flat)


def _tc_body(mask_ref, node_ref, a_ref, we_ref, ws1_ref, wm1_ref,
             ws2_ref, wm2_ref, cent_ref, wout_ref, bout_ref, out_ref):
    step = pl.program_id(0)
    cent = cent_ref[...]
    centf = cent.astype(jnp.float32)
    c2 = jnp.sum(centf * centf, axis=0, keepdims=True)
    iota_col = lax.broadcasted_iota(jnp.int32, (_N, 1), 0)

    nms = [(iota_col < mask_ref[step * _G + g, 0]).astype(jnp.float32)
           for g in range(_G)]
    nm_all = jnp.concatenate(nms, axis=0)

    node = node_ref[...].reshape(_G * _N, _F).astype(jnp.bfloat16)
    h = jnp.dot(node, we_ref[...], preferred_element_type=jnp.float32)
    h = h * nm_all
    for ws, wm in ((ws1_ref, wm1_ref), (ws2_ref, wm2_ref)):
        h16 = h.astype(jnp.bfloat16)
        aggs = [jnp.dot(a_ref[g].astype(jnp.bfloat16),
                        h16[g * _N:(g + 1) * _N],
                        preferred_element_type=jnp.float32)
                for g in range(_G)]
        agg16 = jnp.concatenate(aggs, axis=0).astype(jnp.bfloat16)
        h = jnp.maximum(
            jnp.dot(h16, ws[...], preferred_element_type=jnp.float32)
            + jnp.dot(agg16, wm[...], preferred_element_type=jnp.float32),
            0.0) * nm_all

    x2 = jnp.sum(h * h, axis=1, keepdims=True)
    d2 = x2 + c2 - 2.0 * jnp.dot(h.astype(jnp.bfloat16), cent,
                                 preferred_element_type=jnp.float32)
    dist = jnp.sqrt(jnp.maximum(d2, 1e-6)) * nm_all
    for g in range(_G):
        m = mask_ref[step * _G + g, 0].astype(jnp.float32)
        gr = jnp.sum(dist[g * _N:(g + 1) * _N], axis=0, keepdims=True) / m
        out_ref[g] = jnp.dot(gr, wout_ref[...],
                             preferred_element_type=jnp.float32) + bout_ref[...]


def kernel(node, adj, weight, mask, W_embed, W_self1, W_msg1, W_self2,
           W_msg2, centroids, W_out, b_out):
    adj_flat = adj.astype(jnp.int32).reshape(_B, _N * _K)
    w_flat = weight.reshape(_B, _N * _K)
    mask32 = mask.astype(jnp.int32)
    bf = jnp.bfloat16
    weT = W_embed.T.astype(bf)
    ws1T, wm1T = W_self1.T.astype(bf), W_msg1.T.astype(bf)
    ws2T, wm2T = W_self2.T.astype(bf), W_msg2.T.astype(bf)
    centP = jnp.zeros((_D, _CP), bf).at[:, :_C].set(centroids.T.astype(bf))
    woutP = jnp.zeros((_CP, _CP), jnp.float32).at[:_C, :_NC].set(W_out.T)
    boutP = jnp.zeros((1, _CP), jnp.float32).at[0, :_NC].set(b_out)

    a_dense = _sc_build(adj_flat, w_flat)

    rep = lambda b: (0, 0)
    out = pl.pallas_call(
        _tc_body,
        grid=(_B // _G,),
        in_specs=[
            pl.BlockSpec(memory_space=pltpu.SMEM),
            pl.BlockSpec((_G, _N, _F), lambda b: (b, 0, 0)),
            pl.BlockSpec((_G, _N, _N), lambda b: (b, 0, 0)),
            pl.BlockSpec((_F, _D), rep),
            pl.BlockSpec((_D, _D), rep),
            pl.BlockSpec((_D, _D), rep),
            pl.BlockSpec((_D, _D), rep),
            pl.BlockSpec((_D, _D), rep),
            pl.BlockSpec((_D, _CP), rep),
            pl.BlockSpec((_CP, _CP), rep),
            pl.BlockSpec((1, _CP), rep),
        ],
        out_specs=pl.BlockSpec((_G, 1, _CP), lambda b: (b, 0, 0)),
        out_shape=jax.ShapeDtypeStruct((_B, 1, _CP), jnp.float32),
    )(mask32, node, a_dense, weT, ws1T, wm1T, ws2T, wm2T,
      centP, woutP, boutP)
    return out[:, 0, :_NC]


# speedup vs baseline: 170.1499x; 1.0330x over previous
import functools

import jax
import jax.numpy as jnp
from jax import lax
from jax.experimental import pallas as pl
from jax.experimental.pallas import tpu as pltpu
from jax.experimental.pallas import tpu_sc as plsc

_B, _N, _K, _F, _D, _C, _NC = 32, 512, 32, 256, 256, 100, 10
_CP = 128
_G = 4
_RC = 64
_NCORES, _NSUB, _L = 2, 16, 16


def _sc_build_body(adj_hbm, w_hbm, a_hbm, adj_v, w_v, chunk0, chunk1,
                   sem0, sem1):
    b = lax.axis_index("s") * _NCORES + lax.axis_index("c")
    pltpu.sync_copy(adj_hbm.at[b], adj_v)
    pltpu.sync_copy(w_hbm.at[b], w_v)
    zero16 = jnp.zeros((_L,), jnp.float32)
    iota16 = lax.iota(jnp.int32, _L)
    iotaK = iota16 * _K
    chunks = (chunk0, chunk1)
    sems = (sem0, sem1)
    copies = []
    for c in range(_N // _RC):
        chunk = chunks[c % 2]
        if c >= 2:
            copies[c - 2].wait()

        def zrow(i, carry):
            for j in range(_N // _L):
                chunk[i, pl.ds(j * _L, _L)] = zero16
            return carry
        lax.fori_loop(0, _RC, zrow, 0)

        def srow(rg, carry):
            row0 = c * _RC + rg * _L
            gbase = iotaK + row0 * _K
            rows = iota16 + rg * _L
            for k in range(_K):
                cols = plsc.load_gather(adj_v, [gbase + k])
                vals = plsc.load_gather(w_v, [gbase + k])
                plsc.addupdate_scatter(chunk, [rows, cols], vals)
            return carry
        lax.fori_loop(0, _RC // _L, srow, 0)

        cp = pltpu.make_async_copy(
            chunk, a_hbm.at[b, pl.ds(c * _RC, _RC)], sems[c % 2])
        cp.start()
        copies.append(cp)
    copies[-2].wait()
    copies[-1].wait()


def _sc_build(adj_flat, w_flat):
    built = functools.partial(
        pl.kernel,
        out_type=jax.ShapeDtypeStruct((_B, _N, _N), jnp.float32),
        mesh=plsc.VectorSubcoreMesh(core_axis_name="c", subcore_axis_name="s",
                                    num_cores=_NCORES, num_subcores=_NSUB),
        compiler_params=pltpu.CompilerParams(needs_layout_passes=False),
        scratch_types=[
            pltpu.VMEM((_N * _K,), jnp.int32),
            pltpu.VMEM((_N * _K,), jnp.float32),
            pltpu.VMEM((_RC, _N), jnp.float32),
            pltpu.VMEM((_RC, _N), jnp.float32),
            pltpu.SemaphoreType.DMA,
            pltpu.SemaphoreType.DMA,
        ],
    )(_sc_build_body)
    return built(adj_flat, w_flat)


def _embed_body(mask_ref, node_ref, we_ref, h0_ref):
    step = pl.program_id(0)
    iota_col = lax.broadcasted_iota(jnp.int32, (_N, 1), 0)
    nms = [(iota_col < mask_ref[step * _G + g, 0]).astype(jnp.float32)
           for g in range(_G)]
    nm_all = jnp.concatenate(nms, axis=0)
    node = node_ref[...].reshape(_G * _N, _F).astype(jnp.bfloat16)
    h = jnp.dot(node, we_ref[...], preferred_element_type=jnp.float32)
    h0_ref[...] = (h * nm_all).astype(jnp.bfloat16).reshape(_G, _N, _D)


def _tc_body(mask_ref, h0_ref, a_ref, ws1_ref, wm1_ref,
             ws2_ref, wm2_ref, cent_ref, wout_ref, bout_ref, out_ref):
    step = pl.program_id(0)
    cent = cent_ref[...]
    centf = cent.astype(jnp.float32)
    c2 = jnp.sum(centf * centf, axis=0, keepdims=True)
    iota_col = lax.broadcasted_iota(jnp.int32, (_N, 1), 0)

    nms = [(iota_col < mask_ref[step * _G + g, 0]).astype(jnp.float32)
           for g in range(_G)]
    nm_all = jnp.concatenate(nms, axis=0)

    h16 = h0_ref[...].reshape(_G * _N, _D)
    first = True
    for ws, wm in ((ws1_ref, wm1_ref), (ws2_ref, wm2_ref)):
        if not first:
            h16 = h.astype(jnp.bfloat16)
        first = False
        aggs = [jnp.dot(a_ref[g].astype(jnp.bfloat16),
                        h16[g * _N:(g + 1) * _N],
                        preferred_element_type=jnp.float32)
                for g in range(_G)]
        agg16 = jnp.concatenate(aggs, axis=0).astype(jnp.bfloat16)
        h = jnp.maximum(
            jnp.dot(h16, ws[...], preferred_element_type=jnp.float32)
            + jnp.dot(agg16, wm[...], preferred_element_type=jnp.float32),
            0.0) * nm_all

    x2 = jnp.sum(h * h, axis=1, keepdims=True)
    d2 = x2 + c2 - 2.0 * jnp.dot(h.astype(jnp.bfloat16), cent,
                                 preferred_element_type=jnp.float32)
    dist = jnp.sqrt(jnp.maximum(d2, 1e-6)) * nm_all
    for g in range(_G):
        m = mask_ref[step * _G + g, 0].astype(jnp.float32)
        gr = jnp.sum(dist[g * _N:(g + 1) * _N], axis=0, keepdims=True) / m
        out_ref[g] = jnp.dot(gr, wout_ref[...],
                             preferred_element_type=jnp.float32) + bout_ref[...]


def kernel(node, adj, weight, mask, W_embed, W_self1, W_msg1, W_self2,
           W_msg2, centroids, W_out, b_out):
    adj_flat = adj.astype(jnp.int32).reshape(_B, _N * _K)
    w_flat = weight.reshape(_B, _N * _K)
    mask32 = mask.astype(jnp.int32)
    bf = jnp.bfloat16
    weT = W_embed.T.astype(bf)
    ws1T, wm1T = W_self1.T.astype(bf), W_msg1.T.astype(bf)
    ws2T, wm2T = W_self2.T.astype(bf), W_msg2.T.astype(bf)
    centP = jnp.zeros((_D, _CP), bf).at[:, :_C].set(centroids.T.astype(bf))
    woutP = jnp.zeros((_CP, _CP), jnp.float32).at[:_C, :_NC].set(W_out.T)
    boutP = jnp.zeros((1, _CP), jnp.float32).at[0, :_NC].set(b_out)

    a_dense = _sc_build(adj_flat, w_flat)

    rep = lambda b: (0, 0)
    h0 = pl.pallas_call(
        _embed_body,
        grid=(_B // _G,),
        in_specs=[
            pl.BlockSpec(memory_space=pltpu.SMEM),
            pl.BlockSpec((_G, _N, _F), lambda b: (b, 0, 0)),
            pl.BlockSpec((_F, _D), rep),
        ],
        out_specs=pl.BlockSpec((_G, _N, _D), lambda b: (b, 0, 0)),
        out_shape=jax.ShapeDtypeStruct((_B, _N, _D), jnp.bfloat16),
    )(mask32, node, weT)

    out = pl.pallas_call(
        _tc_body,
        grid=(_B // _G,),
        in_specs=[
            pl.BlockSpec(memory_space=pltpu.SMEM),
            pl.BlockSpec((_G, _N, _D), lambda b: (b, 0, 0)),
            pl.BlockSpec((_G, _N, _N), lambda b: (b, 0, 0)),
            pl.BlockSpec((_D, _D), rep),
            pl.BlockSpec((_D, _D), rep),
            pl.BlockSpec((_D, _D), rep),
            pl.BlockSpec((_D, _D), rep),
            pl.BlockSpec((_D, _CP), rep),
            pl.BlockSpec((_CP, _CP), rep),
            pl.BlockSpec((1, _CP), rep),
        ],
        out_specs=pl.BlockSpec((_G, 1, _CP), lambda b: (b, 0, 0)),
        out_shape=jax.ShapeDtypeStruct((_B, 1, _CP), jnp.float32),
    )(mask32, h0, a_dense, ws1T, wm1T, ws2T, wm2T,
      centP, woutP, boutP)
    return out[:, 0, :_NC]
